# baseline (speedup 1.0000x reference)
import functools

import jax
import jax.numpy as jnp
from jax import lax
from jax.experimental import pallas as pl
from jax.experimental.pallas import tpu as pltpu
from jax.experimental.pallas import tpu_sc as plsc

N = 10000
NP = 10240
E = 640000
NC = 2
NS = 16
CH = 160
PTE = CH * 128
E_PAD = NC * NS * PTE
NPT = NP // NS


def _sc_agg(D):
  mesh = plsc.VectorSubcoreMesh(core_axis_name="c", subcore_axis_name="s",
                                num_cores=NC, num_subcores=NS)
  grp = D // 16

  def body(table, srcr, dstr, wr, out, src_v, dst_v, w_v, r0, r1, bounce,
           acc, g0, g1):
    rbufs = (r0, r1)
    gsems = (g0, g1)
    c = lax.axis_index("c")
    s = lax.axis_index("s")
    wid = c * NS + s

    @plsc.parallel_loop(0, NPT, unroll=8)
    def _(i):
      for j in range(grp):
        bounce[i, pl.ds(j * 16, 16)] = jnp.zeros((16,), jnp.float32)

    pltpu.sync_copy(bounce, acc.at[pl.ds(s * NPT, NPT)])

    pltpu.sync_copy(srcr.at[wid], src_v)
    pltpu.sync_copy(dstr.at[wid], dst_v)
    pltpu.sync_copy(wr.at[wid], w_v)
    plsc.subcore_barrier()

    def scale(rows, g):
      @plsc.parallel_loop(0, 8, unroll=2)
      def _(q):
        wv = w_v[g, pl.ds(q * 16, 16)]
        for i in range(16):
          we = wv[i]
          for j in range(grp):
            idx = (q * 16 + i, pl.ds(j * 16, 16))
            rows[idx] = rows[idx] * we

    def gather(g, p):
      pltpu.async_copy(table.at[src_v.at[g]], rbufs[p], gsems[p])

    def gather_wait(g, p):
      pltpu.make_async_copy(table.at[src_v.at[g]], rbufs[p],
                            gsems[p]).wait()


    gather(0, 0)

    def step(t, _):
      g0 = 2 * t
      gather(g0 + 1, 1)
      gather_wait(g0, 0)
      scale(rbufs[0], g0)
      pltpu.sync_copy(rbufs[0], acc.at[dst_v.at[g0]], add=True)

      @pl.when(g0 + 2 < CH)
      def _():
        gather(g0 + 2, 0)

      gather_wait(g0 + 1, 1)
      scale(rbufs[1], g0 + 1)
      pltpu.sync_copy(rbufs[1], acc.at[dst_v.at[g0 + 1]], add=True)
      return 0

    lax.fori_loop(0, CH // 2, step, 0)
    plsc.subcore_barrier()

    pltpu.sync_copy(acc.at[pl.ds(s * NPT, NPT)], bounce)
    pltpu.sync_copy(bounce, out.at[c, pl.ds(s * NPT, NPT)])

  return pl.kernel(
      body,
      out_type=jax.ShapeDtypeStruct((NC, NP, D), jnp.float32),
      mesh=mesh,
      compiler_params=pltpu.CompilerParams(use_tc_tiling_on_sc=False),
      scratch_types=[
          pltpu.VMEM((CH, 128), jnp.int32),
          pltpu.VMEM((CH, 128), jnp.int32),
          pltpu.VMEM((CH, 128), jnp.float32),
          pltpu.VMEM((128, D), jnp.float32),
          pltpu.VMEM((128, D), jnp.float32),
          pltpu.VMEM((NPT, D), jnp.float32),
          pltpu.VMEM_SHARED((NP, D), jnp.float32),
          pltpu.SemaphoreType.DMA,
          pltpu.SemaphoreType.DMA,
      ],
  )


def _tc1(agg0, agg1, xp, w1rel, b1r, w1root, w2rel, w2root, b2r):
  def body(a0, a1, x_r, wr_r, b1_r, wo_r, w2r_r, w2o_r, b2_r, h2_o, hr_o):
    agg = a0[...] + a1[...]
    h = jnp.dot(agg, wr_r[...], preferred_element_type=jnp.float32)
    h += jnp.dot(x_r[...], wo_r[...], preferred_element_type=jnp.float32)
    h = jnp.maximum(h + b1_r[...], 0.0)
    h2_o[...] = jnp.dot(h, w2r_r[...], preferred_element_type=jnp.float32)
    hr_o[...] = (jnp.dot(h, w2o_r[...], preferred_element_type=jnp.float32)
                 + b2_r[...])

  return pl.pallas_call(
      body,
      out_shape=[
          jax.ShapeDtypeStruct((NP, 16), jnp.float32),
          jax.ShapeDtypeStruct((NP, 8), jnp.float32),
      ],
  )(agg0, agg1, xp, w1rel, b1r, w1root, w2rel, w2root, b2r)


def _tc2(p0, p1, hr):
  def body(p0_r, p1_r, hr_r, o_r):
    o_r[...] = jnp.maximum(p0_r[:, :8] + p1_r[:, :8] + hr_r[...], 0.0)

  return pl.pallas_call(
      body,
      out_shape=jax.ShapeDtypeStruct((NP, 8), jnp.float32),
  )(p0, p1, hr)


def kernel(x, edge_index, edge_weight, W1_rel, b1, W1_root, W2_rel, b2,
           W2_root):
  xp = jnp.pad(x, ((0, NP - N), (0, 32 - x.shape[1])))
  pad = E_PAD - E
  srcr = jnp.pad(edge_index[0], (0, pad)).reshape(NC * NS, CH, 128)
  dstr = jnp.pad(edge_index[1], (0, pad)).reshape(NC * NS, CH, 128)
  wr = jnp.pad(edge_weight, (0, pad)).reshape(NC * NS, CH, 128)

  agg1 = _sc_agg(32)(xp, srcr, dstr, wr)

  w1rel = jnp.pad(W1_rel.T, ((0, 3), (0, 0)))
  w1root = jnp.pad(W1_root.T, ((0, 3), (0, 0)))
  w2rel = jnp.pad(W2_rel.T, ((0, 0), (0, 8)))
  h2p, hr = _tc1(agg1[0], agg1[1], xp, w1rel, b1[None], w1root, w2rel,
                 W2_root.T, b2[None])

  agg2 = _sc_agg(16)(h2p, srcr, dstr, wr)
  return _tc2(agg2[0], agg2[1], hr)[:N]


# speedup vs baseline: 1.4775x; 1.4775x over previous
import functools

import jax
import jax.numpy as jnp
from jax import lax
from jax.experimental import pallas as pl
from jax.experimental.pallas import tpu as pltpu
from jax.experimental.pallas import tpu_sc as plsc

N = 10000
NP = 10240
E = 640000
NC = 2
NS = 16
CH = 158
PTE = CH * 128
E_PAD = NC * NS * PTE
NPT = NP // NS


def _sc_agg(D):
  mesh = plsc.VectorSubcoreMesh(core_axis_name="c", subcore_axis_name="s",
                                num_cores=NC, num_subcores=NS)
  grp = D // 16

  def body(table, srcr, dstr, wr, out, src_v, dst_v, w_v, r0, r1, r2, r3,
           bounce, acc, g0, g1, g2, g3, s0, s1, s2, s3):
    rbufs = (r0, r1, r2, r3)
    gsems = (g0, g1, g2, g3)
    ssems = (s0, s1, s2, s3)
    c = lax.axis_index("c")
    s = lax.axis_index("s")
    wid = c * NS + s

    @plsc.parallel_loop(0, NPT, unroll=8)
    def _(i):
      for j in range(grp):
        bounce[i, pl.ds(j * 16, 16)] = jnp.zeros((16,), jnp.float32)

    pltpu.sync_copy(bounce, acc.at[pl.ds(s * NPT, NPT)])

    pltpu.sync_copy(srcr.at[wid], src_v)
    pltpu.sync_copy(dstr.at[wid], dst_v)
    pltpu.sync_copy(wr.at[wid], w_v)
    plsc.subcore_barrier()

    def scale(rows, g):
      @plsc.parallel_loop(0, 8, unroll=2)
      def _(q):
        wv = w_v[g, pl.ds(q * 16, 16)]
        for i in range(16):
          we = wv[i]
          for j in range(grp):
            idx = (q * 16 + i, pl.ds(j * 16, 16))
            rows[idx] = rows[idx] * we

    def gather(g, p):
      pltpu.async_copy(table.at[src_v.at[g]], rbufs[p], gsems[p])

    def gather_wait(g, p):
      pltpu.make_async_copy(table.at[src_v.at[g]], rbufs[p],
                            gsems[p]).wait()

    def scatter(g, p):
      pltpu.async_copy(rbufs[p], acc.at[dst_v.at[g]], ssems[p], add=True)

    def scatter_wait(g, p):
      pltpu.make_async_copy(rbufs[p], acc.at[dst_v.at[g]], ssems[p]).wait()


    gather(0, 0)
    gather(1, 1)

    def chunk(g, i, tail):
      pn = (i + 2) % 4

      @pl.when(g >= 2)
      def _():
        scatter_wait(g - 2, pn)

      if not tail:
        @pl.when(g + 2 < CH)
        def _():
          gather(g + 2, pn)

      gather_wait(g, i)
      scale(rbufs[i], g)
      scatter(g, i)

    def step(t, _):
      for i in range(4):
        chunk(4 * t + i, i, False)
      return 0

    lax.fori_loop(0, CH // 4, step, 0)
    for g in range(4 * (CH // 4), CH):
      chunk(g, g % 4, True)
    scatter_wait(CH - 2, (CH - 2) % 4)
    scatter_wait(CH - 1, (CH - 1) % 4)
    plsc.subcore_barrier()

    pltpu.sync_copy(acc.at[pl.ds(s * NPT, NPT)], bounce)
    pltpu.sync_copy(bounce, out.at[c, pl.ds(s * NPT, NPT)])

  return pl.kernel(
      body,
      out_type=jax.ShapeDtypeStruct((NC, NP, D), jnp.float32),
      mesh=mesh,
      compiler_params=pltpu.CompilerParams(use_tc_tiling_on_sc=False),
      scratch_types=[
          pltpu.VMEM((CH, 128), jnp.int32),
          pltpu.VMEM((CH, 128), jnp.int32),
          pltpu.VMEM((CH, 128), jnp.float32),
          pltpu.VMEM((128, D), jnp.float32),
          pltpu.VMEM((128, D), jnp.float32),
          pltpu.VMEM((128, D), jnp.float32),
          pltpu.VMEM((128, D), jnp.float32),
          pltpu.VMEM((NPT, D), jnp.float32),
          pltpu.VMEM_SHARED((NP, D), jnp.float32),
          pltpu.SemaphoreType.DMA,
          pltpu.SemaphoreType.DMA,
          pltpu.SemaphoreType.DMA,
          pltpu.SemaphoreType.DMA,
          pltpu.SemaphoreType.DMA,
          pltpu.SemaphoreType.DMA,
          pltpu.SemaphoreType.DMA,
          pltpu.SemaphoreType.DMA,
      ],
  )


def _tc1(agg0, agg1, xp, w1rel, b1r, w1root, w2rel, w2root, b2r):
  def body(a0, a1, x_r, wr_r, b1_r, wo_r, w2r_r, w2o_r, b2_r, h2_o, hr_o):
    agg = a0[...] + a1[...]
    h = jnp.dot(agg, wr_r[...], preferred_element_type=jnp.float32)
    h += jnp.dot(x_r[...], wo_r[...], preferred_element_type=jnp.float32)
    h = jnp.maximum(h + b1_r[...], 0.0)
    h2_o[...] = jnp.dot(h, w2r_r[...], preferred_element_type=jnp.float32)
    hr_o[...] = (jnp.dot(h, w2o_r[...], preferred_element_type=jnp.float32)
                 + b2_r[...])

  return pl.pallas_call(
      body,
      out_shape=[
          jax.ShapeDtypeStruct((NP, 16), jnp.float32),
          jax.ShapeDtypeStruct((NP, 8), jnp.float32),
      ],
  )(agg0, agg1, xp, w1rel, b1r, w1root, w2rel, w2root, b2r)


def _tc2(p0, p1, hr):
  def body(p0_r, p1_r, hr_r, o_r):
    o_r[...] = jnp.maximum(p0_r[:, :8] + p1_r[:, :8] + hr_r[...], 0.0)

  return pl.pallas_call(
      body,
      out_shape=jax.ShapeDtypeStruct((NP, 8), jnp.float32),
  )(p0, p1, hr)


def kernel(x, edge_index, edge_weight, W1_rel, b1, W1_root, W2_rel, b2,
           W2_root):
  xp = jnp.pad(x, ((0, NP - N), (0, 32 - x.shape[1])))
  pad = E_PAD - E
  srcr = jnp.pad(edge_index[0], (0, pad)).reshape(NC * NS, CH, 128)
  dstr = jnp.pad(edge_index[1], (0, pad)).reshape(NC * NS, CH, 128)
  wr = jnp.pad(edge_weight, (0, pad)).reshape(NC * NS, CH, 128)

  agg1 = _sc_agg(32)(xp, srcr, dstr, wr)

  w1rel = jnp.pad(W1_rel.T, ((0, 3), (0, 0)))
  w1root = jnp.pad(W1_root.T, ((0, 3), (0, 0)))
  w2rel = jnp.pad(W2_rel.T, ((0, 0), (0, 8)))
  h2p, hr = _tc1(agg1[0], agg1[1], xp, w1rel, b1[None], w1root, w2rel,
                 W2_root.T, b2[None])

  agg2 = _sc_agg(16)(h2p, srcr, dstr, wr)
  return _tc2(agg2[0], agg2[1], hr)[:N]


# speedup vs baseline: 1.6255x; 1.1002x over previous
import jax
import jax.numpy as jnp
from jax import lax
from jax.experimental import pallas as pl
from jax.experimental.pallas import tpu as pltpu
from jax.experimental.pallas import tpu_sc as plsc

N = 10000
NP = 10240
E = 640000
NC = 2
NS = 16
CH0 = 105
CH1 = 209
CHM = CH1
TOTCH = NS * (CH0 + CH1)
E_PAD = TOTCH * 128
NPT = NP // NS


def _sc_agg(D):
  mesh = plsc.VectorSubcoreMesh(core_axis_name="c", subcore_axis_name="s",
                                num_cores=NC, num_subcores=NS)
  grp = D // 16

  def body(table, srcr, dstr, wr, out, src_v, dst_v, w_v, r0, r1, r2, r3,
           acc, g0, g1, g2, g3, s0, s1, s2, s3):
    rbufs = (r0, r1, r2, r3)
    gsems = (g0, g1, g2, g3)
    ssems = (s0, s1, s2, s3)
    c = lax.axis_index("c")
    s = lax.axis_index("s")
    cnt = jnp.where(c == 0, CH0, CH1)
    start = jnp.where(c == 0, s * CH0, NS * CH0 + s * CH1)

    @plsc.parallel_loop(0, 128, unroll=8)
    def _(i):
      for j in range(grp):
        r0[i, pl.ds(j * 16, 16)] = jnp.zeros((16,), jnp.float32)

    for k in range(NPT // 128):
      pltpu.sync_copy(r0, acc.at[pl.ds(s * NPT + k * 128, 128)])

    pltpu.sync_copy(srcr.at[pl.ds(start, CHM)], src_v)
    pltpu.sync_copy(dstr.at[pl.ds(start, CHM)], dst_v)
    pltpu.sync_copy(wr.at[pl.ds(start, CHM)], w_v)
    plsc.subcore_barrier()

    def scale(rows, g):
      @plsc.parallel_loop(0, 8, unroll=2)
      def _(q):
        wv = w_v[g, pl.ds(q * 16, 16)]
        for i in range(16):
          we = wv[i]
          for j in range(grp):
            idx = (q * 16 + i, pl.ds(j * 16, 16))
            rows[idx] = rows[idx] * we

    def gather(g, p):
      pltpu.async_copy(table.at[src_v.at[g]], rbufs[p], gsems[p])

    def gather_wait(g, p):
      pltpu.make_async_copy(table.at[src_v.at[g]], rbufs[p],
                            gsems[p]).wait()

    def scatter(g, p):
      pltpu.async_copy(rbufs[p], acc.at[dst_v.at[g]], ssems[p], add=True)

    def scatter_wait(g, p):
      pltpu.make_async_copy(rbufs[p], acc.at[dst_v.at[g]], ssems[p]).wait()

    gather(0, 0)
    gather(1, 1)

    def chunk(g, i, tail):
      pn = (i + 2) % 4

      @pl.when(g >= 2)
      def _():
        scatter_wait(g - 2, pn)

      if not tail:
        @pl.when(g + 2 < cnt)
        def _():
          gather(g + 2, pn)

      gather_wait(g, i)
      scale(rbufs[i], g)
      scatter(g, i)

    def step(t, _):
      for i in range(4):
        chunk(4 * t + i, i, False)
      return 0

    lax.fori_loop(0, cnt // 4, step, 0)
    chunk(cnt - 1, 0, True)
    scatter_wait(cnt - 2, 3)
    scatter_wait(cnt - 1, 0)
    plsc.subcore_barrier()

    for k in range(NPT // 128):
      pltpu.sync_copy(acc.at[pl.ds(s * NPT + k * 128, 128)], rbufs[k % 2])
      pltpu.sync_copy(rbufs[k % 2], out.at[c, pl.ds(s * NPT + k * 128, 128)])

  return pl.kernel(
      body,
      out_type=jax.ShapeDtypeStruct((NC, NP, D), jnp.float32),
      mesh=mesh,
      compiler_params=pltpu.CompilerParams(use_tc_tiling_on_sc=False),
      scratch_types=[
          pltpu.VMEM((CHM, 128), jnp.int32),
          pltpu.VMEM((CHM, 128), jnp.int32),
          pltpu.VMEM((CHM, 128), jnp.float32),
          pltpu.VMEM((128, D), jnp.float32),
          pltpu.VMEM((128, D), jnp.float32),
          pltpu.VMEM((128, D), jnp.float32),
          pltpu.VMEM((128, D), jnp.float32),
          pltpu.VMEM_SHARED((NP, D), jnp.float32),
          pltpu.SemaphoreType.DMA,
          pltpu.SemaphoreType.DMA,
          pltpu.SemaphoreType.DMA,
          pltpu.SemaphoreType.DMA,
          pltpu.SemaphoreType.DMA,
          pltpu.SemaphoreType.DMA,
          pltpu.SemaphoreType.DMA,
          pltpu.SemaphoreType.DMA,
      ],
  )


def _tc1(agg0, agg1, xp, w1rel, b1r, w1root, w2rel, w2root, b2r):
  def body(a0, a1, x_r, wr_r, b1_r, wo_r, w2r_r, w2o_r, b2_r, h2_o, hr_o):
    agg = a0[...] + a1[...]
    h = jnp.dot(agg, wr_r[...], preferred_element_type=jnp.float32)
    h += jnp.dot(x_r[...], wo_r[...], preferred_element_type=jnp.float32)
    h = jnp.maximum(h + b1_r[...], 0.0)
    h2_o[...] = jnp.dot(h, w2r_r[...], preferred_element_type=jnp.float32)
    hr_o[...] = (jnp.dot(h, w2o_r[...], preferred_element_type=jnp.float32)
                 + b2_r[...])

  return pl.pallas_call(
      body,
      out_shape=[
          jax.ShapeDtypeStruct((NP, 16), jnp.float32),
          jax.ShapeDtypeStruct((NP, 8), jnp.float32),
      ],
  )(agg0, agg1, xp, w1rel, b1r, w1root, w2rel, w2root, b2r)


def _tc2(p0, p1, hr):
  def body(p0_r, p1_r, hr_r, o_r):
    o_r[...] = jnp.maximum(p0_r[:, :8] + p1_r[:, :8] + hr_r[...], 0.0)

  return pl.pallas_call(
      body,
      out_shape=jax.ShapeDtypeStruct((NP, 8), jnp.float32),
  )(p0, p1, hr)


def kernel(x, edge_index, edge_weight, W1_rel, b1, W1_root, W2_rel, b2,
           W2_root):
  xp = jnp.pad(x, ((0, NP - N), (0, 32 - x.shape[1])))
  pad = E_PAD - E
  srcr = jnp.pad(edge_index[0], (0, pad)).reshape(TOTCH, 128)
  dstr = jnp.pad(edge_index[1], (0, pad)).reshape(TOTCH, 128)
  wr = jnp.pad(edge_weight, (0, pad)).reshape(TOTCH, 128)

  agg1 = _sc_agg(32)(xp, srcr, dstr, wr)

  w1rel = jnp.pad(W1_rel.T, ((0, 3), (0, 0)))
  w1root = jnp.pad(W1_root.T, ((0, 3), (0, 0)))
  w2rel = jnp.pad(W2_rel.T, ((0, 0), (0, 8)))
  h2p, hr = _tc1(agg1[0], agg1[1], xp, w1rel, b1[None], w1root, w2rel,
                 W2_root.T, b2[None])

  agg2 = _sc_agg(16)(h2p, srcr, dstr, wr)
  return _tc2(agg2[0], agg2[1], hr)[:N]


# speedup vs baseline: 1.8468x; 1.1362x over previous
import jax
import jax.numpy as jnp
from jax import lax
from jax.experimental import pallas as pl
from jax.experimental.pallas import tpu as pltpu
from jax.experimental.pallas import tpu_sc as plsc

N = 10000
NP = 10240
E = 640000
NC = 2
NS = 16
CH0 = 169
CH1 = 145
CHM = max(CH0, CH1)
TOTCH = NS * (CH0 + CH1) + CHM
E_PAD = TOTCH * 128
NPT = NP // NS


def _sc_agg(D):
  mesh = plsc.VectorSubcoreMesh(core_axis_name="c", subcore_axis_name="s",
                                num_cores=NC, num_subcores=NS)
  grp = D // 16

  def body(table, srcr, dstr, wr, out, src_v, dst_v, w_v, r0, r1, r2, r3,
           acc, g0, g1, g2, g3, s0, s1, s2, s3):
    rbufs = (r0, r1, r2, r3)
    gsems = (g0, g1, g2, g3)
    ssems = (s0, s1, s2, s3)
    c = lax.axis_index("c")
    s = lax.axis_index("s")
    cnt = jnp.where(c == 0, CH0, CH1)
    start = jnp.where(c == 0, s * CH0, NS * CH0 + s * CH1)

    @plsc.parallel_loop(0, 128, unroll=8)
    def _(i):
      for j in range(grp):
        r0[i, pl.ds(j * 16, 16)] = jnp.zeros((16,), jnp.float32)

    for k in range(NPT // 128):
      pltpu.sync_copy(r0, acc.at[pl.ds(s * NPT + k * 128, 128)])

    pltpu.sync_copy(srcr.at[pl.ds(start, CHM)], src_v)
    pltpu.sync_copy(dstr.at[pl.ds(start, CHM)], dst_v)
    pltpu.sync_copy(wr.at[pl.ds(start, CHM)], w_v)
    plsc.subcore_barrier()

    def scale(rows, g):
      @plsc.parallel_loop(0, 8, unroll=2)
      def _(q):
        wv = w_v[g, pl.ds(q * 16, 16)]
        for i in range(16):
          we = wv[i]
          for j in range(grp):
            idx = (q * 16 + i, pl.ds(j * 16, 16))
            rows[idx] = rows[idx] * we

    def gather(g, p):
      pltpu.async_copy(table.at[src_v.at[g]], rbufs[p], gsems[p])

    def gather_wait(g, p):
      pltpu.make_async_copy(table.at[src_v.at[g]], rbufs[p],
                            gsems[p]).wait()

    def scatter(g, p):
      pltpu.async_copy(rbufs[p], acc.at[dst_v.at[g]], ssems[p], add=True)

    def scatter_wait(g, p):
      pltpu.make_async_copy(rbufs[p], acc.at[dst_v.at[g]], ssems[p]).wait()

    gather(0, 0)
    gather(1, 1)

    def chunk(g, i, tail):
      pn = (i + 2) % 4

      @pl.when(g >= 2)
      def _():
        scatter_wait(g - 2, pn)

      if not tail:
        @pl.when(g + 2 < cnt)
        def _():
          gather(g + 2, pn)

      gather_wait(g, i)
      scale(rbufs[i], g)
      scatter(g, i)

    def step(t, _):
      for i in range(4):
        chunk(4 * t + i, i, False)
      return 0

    lax.fori_loop(0, cnt // 4, step, 0)
    chunk(cnt - 1, 0, True)
    scatter_wait(cnt - 2, 3)
    scatter_wait(cnt - 1, 0)
    plsc.subcore_barrier()

    for k in range(NPT // 128):
      pltpu.sync_copy(acc.at[pl.ds(s * NPT + k * 128, 128)], rbufs[k % 2])
      pltpu.sync_copy(rbufs[k % 2], out.at[c, pl.ds(s * NPT + k * 128, 128)])

  return pl.kernel(
      body,
      out_type=jax.ShapeDtypeStruct((NC, NP, D), jnp.float32),
      mesh=mesh,
      compiler_params=pltpu.CompilerParams(use_tc_tiling_on_sc=False),
      scratch_types=[
          pltpu.VMEM((CHM, 128), jnp.int32),
          pltpu.VMEM((CHM, 128), jnp.int32),
          pltpu.VMEM((CHM, 128), jnp.float32),
          pltpu.VMEM((128, D), jnp.float32),
          pltpu.VMEM((128, D), jnp.float32),
          pltpu.VMEM((128, D), jnp.float32),
          pltpu.VMEM((128, D), jnp.float32),
          pltpu.VMEM_SHARED((NP, D), jnp.float32),
          pltpu.SemaphoreType.DMA,
          pltpu.SemaphoreType.DMA,
          pltpu.SemaphoreType.DMA,
          pltpu.SemaphoreType.DMA,
          pltpu.SemaphoreType.DMA,
          pltpu.SemaphoreType.DMA,
          pltpu.SemaphoreType.DMA,
          pltpu.SemaphoreType.DMA,
      ],
  )


def _tc1(agg0, agg1, xp, w1rel, b1r, w1root, w2rel, w2root, b2r):
  def body(a0, a1, x_r, wr_r, b1_r, wo_r, w2r_r, w2o_r, b2_r, h2_o, hr_o):
    agg = a0[...] + a1[...]
    h = jnp.dot(agg, wr_r[...], preferred_element_type=jnp.float32)
    h += jnp.dot(x_r[...], wo_r[...], preferred_element_type=jnp.float32)
    h = jnp.maximum(h + b1_r[...], 0.0)
    h2_o[...] = jnp.dot(h, w2r_r[...], preferred_element_type=jnp.float32)
    hr_o[...] = (jnp.dot(h, w2o_r[...], preferred_element_type=jnp.float32)
                 + b2_r[...])

  return pl.pallas_call(
      body,
      out_shape=[
          jax.ShapeDtypeStruct((NP, 16), jnp.float32),
          jax.ShapeDtypeStruct((NP, 8), jnp.float32),
      ],
  )(agg0, agg1, xp, w1rel, b1r, w1root, w2rel, w2root, b2r)


def _tc2(p0, p1, hr):
  def body(p0_r, p1_r, hr_r, o_r):
    o_r[...] = jnp.maximum(p0_r[:, :8] + p1_r[:, :8] + hr_r[...], 0.0)

  return pl.pallas_call(
      body,
      out_shape=jax.ShapeDtypeStruct((NP, 8), jnp.float32),
  )(p0, p1, hr)


def kernel(x, edge_index, edge_weight, W1_rel, b1, W1_root, W2_rel, b2,
           W2_root):
  xp = jnp.pad(x, ((0, NP - N), (0, 32 - x.shape[1])))
  pad = E_PAD - E
  srcr = jnp.pad(edge_index[0], (0, pad)).reshape(TOTCH, 128)
  dstr = jnp.pad(edge_index[1], (0, pad)).reshape(TOTCH, 128)
  wr = jnp.pad(edge_weight, (0, pad)).reshape(TOTCH, 128)

  agg1 = _sc_agg(32)(xp, srcr, dstr, wr)

  w1rel = jnp.pad(W1_rel.T, ((0, 3), (0, 0)))
  w1root = jnp.pad(W1_root.T, ((0, 3), (0, 0)))
  w2rel = jnp.pad(W2_rel.T, ((0, 0), (0, 8)))
  h2p, hr = _tc1(agg1[0], agg1[1], xp, w1rel, b1[None], w1root, w2rel,
                 W2_root.T, b2[None])

  agg2 = _sc_agg(16)(h2p, srcr, dstr, wr)
  return _tc2(agg2[0], agg2[1], hr)[:N]


# speedup vs baseline: 1.8538x; 1.0038x over previous
import jax
import jax.numpy as jnp
from jax import lax
from jax.experimental import pallas as pl
from jax.experimental.pallas import tpu as pltpu
from jax.experimental.pallas import tpu_sc as plsc

N = 10000
NP = 10240
E = 640000
NC = 2
NS = 16
CH0 = 185
CH1 = 129
CHM = max(CH0, CH1)
TOTCH = NS * (CH0 + CH1) + CHM
E_PAD = TOTCH * 128
NPT = NP // NS


def _sc_agg(D):
  mesh = plsc.VectorSubcoreMesh(core_axis_name="c", subcore_axis_name="s",
                                num_cores=NC, num_subcores=NS)
  grp = D // 16

  def body(table, srcr, dstr, wr, out, src_v, dst_v, w_v, r0, r1, r2, r3,
           acc, g0, g1, g2, g3, s0, s1, s2, s3):
    rbufs = (r0, r1, r2, r3)
    gsems = (g0, g1, g2, g3)
    ssems = (s0, s1, s2, s3)
    c = lax.axis_index("c")
    s = lax.axis_index("s")
    cnt = jnp.where(c == 0, CH0, CH1)
    start = jnp.where(c == 0, s * CH0, NS * CH0 + s * CH1)

    @plsc.parallel_loop(0, 128, unroll=8)
    def _(i):
      for j in range(grp):
        r0[i, pl.ds(j * 16, 16)] = jnp.zeros((16,), jnp.float32)

    for k in range(NPT // 128):
      pltpu.sync_copy(r0, acc.at[pl.ds(s * NPT + k * 128, 128)])

    pltpu.sync_copy(srcr.at[pl.ds(start, CHM)], src_v)
    pltpu.sync_copy(dstr.at[pl.ds(start, CHM)], dst_v)
    pltpu.sync_copy(wr.at[pl.ds(start, CHM)], w_v)
    plsc.subcore_barrier()

    def scale(rows, g):
      @plsc.parallel_loop(0, 8, unroll=2)
      def _(q):
        wv = w_v[g, pl.ds(q * 16, 16)]
        for i in range(16):
          we = wv[i]
          for j in range(grp):
            idx = (q * 16 + i, pl.ds(j * 16, 16))
            rows[idx] = rows[idx] * we

    def gather(g, p):
      pltpu.async_copy(table.at[src_v.at[g]], rbufs[p], gsems[p])

    def gather_wait(g, p):
      pltpu.make_async_copy(table.at[src_v.at[g]], rbufs[p],
                            gsems[p]).wait()

    def scatter(g, p):
      pltpu.async_copy(rbufs[p], acc.at[dst_v.at[g]], ssems[p], add=True)

    def scatter_wait(g, p):
      pltpu.make_async_copy(rbufs[p], acc.at[dst_v.at[g]], ssems[p]).wait()

    gather(0, 0)
    gather(1, 1)

    def chunk(g, i, tail):
      pn = (i + 2) % 4

      @pl.when(g >= 2)
      def _():
        scatter_wait(g - 2, pn)

      if not tail:
        @pl.when(g + 2 < cnt)
        def _():
          gather(g + 2, pn)

      gather_wait(g, i)
      scale(rbufs[i], g)
      scatter(g, i)

    def step(t, _):
      for i in range(4):
        chunk(4 * t + i, i, False)
      return 0

    lax.fori_loop(0, cnt // 4, step, 0)
    chunk(cnt - 1, 0, True)
    scatter_wait(cnt - 2, 3)
    scatter_wait(cnt - 1, 0)
    plsc.subcore_barrier()

    for k in range(NPT // 128):
      pltpu.sync_copy(acc.at[pl.ds(s * NPT + k * 128, 128)], rbufs[k % 2])
      pltpu.sync_copy(rbufs[k % 2], out.at[c, pl.ds(s * NPT + k * 128, 128)])

  return pl.kernel(
      body,
      out_type=jax.ShapeDtypeStruct((NC, NP, D), jnp.float32),
      mesh=mesh,
      compiler_params=pltpu.CompilerParams(use_tc_tiling_on_sc=False),
      scratch_types=[
          pltpu.VMEM((CHM, 128), jnp.int32),
          pltpu.VMEM((CHM, 128), jnp.int32),
          pltpu.VMEM((CHM, 128), jnp.float32),
          pltpu.VMEM((128, D), jnp.float32),
          pltpu.VMEM((128, D), jnp.float32),
          pltpu.VMEM((128, D), jnp.float32),
          pltpu.VMEM((128, D), jnp.float32),
          pltpu.VMEM_SHARED((NP, D), jnp.float32),
          pltpu.SemaphoreType.DMA,
          pltpu.SemaphoreType.DMA,
          pltpu.SemaphoreType.DMA,
          pltpu.SemaphoreType.DMA,
          pltpu.SemaphoreType.DMA,
          pltpu.SemaphoreType.DMA,
          pltpu.SemaphoreType.DMA,
          pltpu.SemaphoreType.DMA,
      ],
  )


def _tc1(agg0, agg1, xp, w1rel, b1r, w1root, w2rel, w2root, b2r):
  def body(a0, a1, x_r, wr_r, b1_r, wo_r, w2r_r, w2o_r, b2_r, h2_o, hr_o):
    agg = a0[...] + a1[...]
    h = jnp.dot(agg, wr_r[...], preferred_element_type=jnp.float32)
    h += jnp.dot(x_r[...], wo_r[...], preferred_element_type=jnp.float32)
    h = jnp.maximum(h + b1_r[...], 0.0)
    h2_o[...] = jnp.dot(h, w2r_r[...], preferred_element_type=jnp.float32)
    hr_o[...] = (jnp.dot(h, w2o_r[...], preferred_element_type=jnp.float32)
                 + b2_r[...])

  return pl.pallas_call(
      body,
      out_shape=[
          jax.ShapeDtypeStruct((NP, 16), jnp.float32),
          jax.ShapeDtypeStruct((NP, 8), jnp.float32),
      ],
  )(agg0, agg1, xp, w1rel, b1r, w1root, w2rel, w2root, b2r)


def _tc2(p0, p1, hr):
  def body(p0_r, p1_r, hr_r, o_r):
    o_r[...] = jnp.maximum(p0_r[:, :8] + p1_r[:, :8] + hr_r[...], 0.0)

  return pl.pallas_call(
      body,
      out_shape=jax.ShapeDtypeStruct((NP, 8), jnp.float32),
  )(p0, p1, hr)


def kernel(x, edge_index, edge_weight, W1_rel, b1, W1_root, W2_rel, b2,
           W2_root):
  xp = jnp.pad(x, ((0, NP - N), (0, 32 - x.shape[1])))
  pad = E_PAD - E
  srcr = jnp.pad(edge_index[0], (0, pad)).reshape(TOTCH, 128)
  dstr = jnp.pad(edge_index[1], (0, pad)).reshape(TOTCH, 128)
  wr = jnp.pad(edge_weight, (0, pad)).reshape(TOTCH, 128)

  agg1 = _sc_agg(32)(xp, srcr, dstr, wr)

  w1rel = jnp.pad(W1_rel.T, ((0, 3), (0, 0)))
  w1root = jnp.pad(W1_root.T, ((0, 3), (0, 0)))
  w2rel = jnp.pad(W2_rel.T, ((0, 0), (0, 8)))
  h2p, hr = _tc1(agg1[0], agg1[1], xp, w1rel, b1[None], w1root, w2rel,
                 W2_root.T, b2[None])

  agg2 = _sc_agg(16)(h2p, srcr, dstr, wr)
  return _tc2(agg2[0], agg2[1], hr)[:N]
